# Initial kernel scaffold; baseline (speedup 1.0000x reference)
import jax
import jax.numpy as jnp
from jax.experimental import pallas as pl


def kernel(x, codebooks):
    raise NotImplementedError("write your pallas kernel here")


# speedup vs baseline: 1.1427x; 1.1427x over previous
import functools

import jax
import jax.numpy as jnp
from jax import lax
from jax.experimental import pallas as pl
from jax.experimental.pallas import tpu as pltpu
from jax.experimental.pallas import tpu_sc as plsc

NQ = 4
K = 8192
D = 256
B = 8
T = 576
N = B * T

TN = 2304
NT = N // TN
KT = 512
NKT = K // KT

SC_CORES = 2
SC_SUBCORES = 16
NW = SC_CORES * SC_SUBCORES
BPW = N // NW
GCHUNK = 72
IPW = (NQ * N) // NW


def _stage_body(r_prev_ref, q_prev_ref, cb_ref, idx_ref, r_out_ref, loss_ref,
                bval, bidx, row2s):
    t = pl.program_id(0)
    k = pl.program_id(1)

    @pl.when(k == 0)
    def _init():
        r = r_prev_ref[...] - q_prev_ref[...]
        r_out_ref[...] = r
        row2 = jnp.sum(r * r, axis=1, keepdims=True)
        row2s[...] = row2
        loss_ref[t, 0] = jnp.sum(row2)

    r = r_out_ref[...]
    cbt = cb_ref[...]
    cb2 = jnp.sum(cbt * cbt, axis=1)
    mm = lax.dot_general(r, cbt, (((1,), (1,)), ((), ())),
                         preferred_element_type=jnp.float32)
    dist = (row2s[...] - 2.0 * mm) + cb2[None, :]
    tmin = jnp.min(dist, axis=1, keepdims=True)
    io = lax.broadcasted_iota(jnp.int32, (TN, KT), 1)
    lidx = jnp.min(jnp.where(dist == tmin, io, KT), axis=1,
                   keepdims=True) + k * KT

    @pl.when(k == 0)
    def _first():
        bval[...] = tmin
        bidx[...] = lidx

    @pl.when(k > 0)
    def _update():
        better = tmin < bval[...]
        bval[...] = jnp.where(better, tmin, bval[...])
        bidx[...] = jnp.where(better, lidx, bidx[...])

    @pl.when(k == NKT - 1)
    def _emit():
        idx_ref[...] = bidx[...]


_stage_call = pl.pallas_call(
    _stage_body,
    grid=(NT, NKT),
    in_specs=[
        pl.BlockSpec((TN, D), lambda t, k: (t, 0)),
        pl.BlockSpec((TN, D), lambda t, k: (t, 0)),
        pl.BlockSpec((KT, D), lambda t, k: (k, 0)),
    ],
    out_specs=[
        pl.BlockSpec((TN, 1), lambda t, k: (t, 0)),
        pl.BlockSpec((TN, D), lambda t, k: (t, 0)),
        pl.BlockSpec((NT, 1), lambda t, k: (0, 0), memory_space=pltpu.SMEM),
    ],
    out_shape=[
        jax.ShapeDtypeStruct((N, 1), jnp.int32),
        jax.ShapeDtypeStruct((N, D), jnp.float32),
        jax.ShapeDtypeStruct((NT, 1), jnp.float32),
    ],
    scratch_shapes=[
        pltpu.VMEM((TN, 1), jnp.float32),
        pltpu.VMEM((TN, 1), jnp.int32),
        pltpu.VMEM((TN, 1), jnp.float32),
    ],
    compiler_params=pltpu.CompilerParams(
        dimension_semantics=("arbitrary", "arbitrary")),
)


def _final_body(x_ref, r3_ref, q3_ref, hist_ref, qout_ref, loss_ref, perp_ref):
    r4 = r3_ref[...] - q3_ref[...]
    qout_ref[...] = x_ref[...] - r4
    loss_ref[0, 0] = jnp.sum(r4 * r4)
    counts = jnp.sum(hist_ref[...], axis=0)
    p = counts * (1.0 / (NQ * N))
    ent = jnp.sum(p * jnp.log(p + 1e-10))
    perp_ref[0, 0] = jnp.exp(-ent)


_final_call = pl.pallas_call(
    _final_body,
    out_shape=[
        jax.ShapeDtypeStruct((N, D), jnp.float32),
        jax.ShapeDtypeStruct((1, 1), jnp.float32),
        jax.ShapeDtypeStruct((1, 1), jnp.float32),
    ],
    out_specs=[
        pl.BlockSpec((N, D), lambda: (0, 0)),
        pl.BlockSpec(memory_space=pltpu.SMEM),
        pl.BlockSpec(memory_space=pltpu.SMEM),
    ],
)

@functools.lru_cache(maxsize=None)
def _get_sc_kernels():
    mesh = plsc.VectorSubcoreMesh(
        core_axis_name="c", subcore_axis_name="s",
        num_cores=SC_CORES, num_subcores=SC_SUBCORES)

    @functools.partial(
        pl.kernel,
        out_type=jax.ShapeDtypeStruct((N, D), jnp.float32),
        mesh=mesh,
        scratch_types=[
            pltpu.VMEM((2, GCHUNK), jnp.int32),
            pltpu.VMEM((BPW, D), jnp.float32),
            pltpu.SemaphoreType.DMA,
        ],
    )
    def _sc_gather(cb_hbm, idx_hbm, out_hbm, idx_v, rows_v, sem):
        wid = lax.axis_index("s") * SC_CORES + lax.axis_index("c")
        pltpu.sync_copy(idx_hbm.at[wid], idx_v)
        cp0 = pltpu.async_copy(cb_hbm.at[idx_v.at[0]],
                               rows_v.at[pl.ds(0, GCHUNK)], sem)
        cp1 = pltpu.async_copy(cb_hbm.at[idx_v.at[1]],
                               rows_v.at[pl.ds(GCHUNK, GCHUNK)], sem)
        cp0.wait()
        cp1.wait()
        pltpu.sync_copy(rows_v, out_hbm.at[pl.ds(wid * BPW, BPW)])

    @functools.partial(
        pl.kernel,
        out_type=jax.ShapeDtypeStruct((NW, K), jnp.float32),
        mesh=mesh,
        scratch_types=[
            pltpu.VMEM((IPW,), jnp.int32),
            pltpu.VMEM((K,), jnp.float32),
        ],
        compiler_params=pltpu.CompilerParams(needs_layout_passes=False),
    )
    def _sc_hist(idx_hbm, out_hbm, idx_v, hist_v):
        wid = lax.axis_index("s") * SC_CORES + lax.axis_index("c")
        pltpu.sync_copy(idx_hbm.at[wid], idx_v)

        zeros16 = jnp.zeros((16,), jnp.float32)

        def _zero(i, carry):
            hist_v[pl.ds(i * 16, 16)] = zeros16
            return carry

        lax.fori_loop(0, K // 16, _zero, 0)

        ones16 = jnp.ones((16,), jnp.float32)

        def _accum(i, carry):
            idx16 = idx_v[pl.ds(i * 16, 16)]
            plsc.addupdate_scatter(hist_v, [idx16], ones16)
            return carry

        lax.fori_loop(0, IPW // 16, _accum, 0)
        pltpu.sync_copy(hist_v, out_hbm.at[wid])

    return _sc_gather, _sc_hist


@jax.jit
def kernel(x, codebooks):
    sc_gather, sc_hist = _get_sc_kernels()
    flat = x.reshape(N, D)
    r = flat
    q = jnp.zeros_like(flat)
    idxs = []
    loss_sums = []
    for i in range(NQ):
        idx_i, r_i, lsum = _stage_call(r, q, codebooks[i])
        if i > 0:
            loss_sums.append(jnp.sum(lsum))
        q = sc_gather(codebooks[i], idx_i.reshape(NW, 2, GCHUNK))
        r = r_i
        idxs.append(idx_i)
    idx_all = jnp.concatenate(idxs, axis=1)
    hist = sc_hist(idx_all.reshape(NW, IPW))
    qout, lsum3, perp = _final_call(flat, r, q, hist)
    loss_sums.append(lsum3[0, 0])
    all_losses = jnp.stack(loss_sums) * (1.0 / (N * D))
    return (qout.reshape(B, T, D),
            idx_all.reshape(B, T, NQ),
            all_losses,
            perp[0, 0])


# speedup vs baseline: 1.1526x; 1.0087x over previous
import functools

import jax
import jax.numpy as jnp
from jax import lax
from jax.experimental import pallas as pl
from jax.experimental.pallas import tpu as pltpu
from jax.experimental.pallas import tpu_sc as plsc

NQ = 4
K = 8192
D = 256
B = 8
T = 576
N = B * T

TN = 2304
NT = N // TN
KT = 512
NKT = K // KT

SC_CORES = 2
SC_SUBCORES = 16
NW = SC_CORES * SC_SUBCORES
BPW = N // NW
GCHUNK = 72
IPW = (NQ * N) // NW


def _stage_body(r_prev_ref, q_prev_ref, cb_ref, idx_ref, r_out_ref, loss_ref,
                bval, bidx, row2s, r2s):
    t = pl.program_id(0)
    k = pl.program_id(1)

    @pl.when(k == 0)
    def _init():
        r = r_prev_ref[...] - q_prev_ref[...]
        r_out_ref[...] = r
        r2s[...] = r + r
        row2 = jnp.sum(r * r, axis=1, keepdims=True)
        row2s[...] = row2
        loss_ref[t, 0] = jnp.sum(row2)

    cbt = cb_ref[...]
    cb2 = jnp.sum(cbt * cbt, axis=1)
    mm2 = lax.dot_general(r2s[...], cbt, (((1,), (1,)), ((), ())),
                          preferred_element_type=jnp.float32)
    dist = (row2s[...] - mm2) + cb2[None, :]
    tmin = jnp.min(dist, axis=1, keepdims=True)
    io = lax.broadcasted_iota(jnp.int32, (TN, KT), 1)
    lidx = jnp.min(jnp.where(dist == tmin, io, KT), axis=1,
                   keepdims=True) + k * KT

    @pl.when(k == 0)
    def _first():
        bval[...] = tmin
        bidx[...] = lidx

    @pl.when(k > 0)
    def _update():
        better = tmin < bval[...]
        bval[...] = jnp.where(better, tmin, bval[...])
        bidx[...] = jnp.where(better, lidx, bidx[...])

    @pl.when(k == NKT - 1)
    def _emit():
        idx_ref[...] = bidx[...]


_stage_call = pl.pallas_call(
    _stage_body,
    grid=(NT, NKT),
    in_specs=[
        pl.BlockSpec((TN, D), lambda t, k: (t, 0)),
        pl.BlockSpec((TN, D), lambda t, k: (t, 0)),
        pl.BlockSpec((KT, D), lambda t, k: (k, 0)),
    ],
    out_specs=[
        pl.BlockSpec((TN, 1), lambda t, k: (t, 0)),
        pl.BlockSpec((TN, D), lambda t, k: (t, 0)),
        pl.BlockSpec((NT, 1), lambda t, k: (0, 0), memory_space=pltpu.SMEM),
    ],
    out_shape=[
        jax.ShapeDtypeStruct((N, 1), jnp.int32),
        jax.ShapeDtypeStruct((N, D), jnp.float32),
        jax.ShapeDtypeStruct((NT, 1), jnp.float32),
    ],
    scratch_shapes=[
        pltpu.VMEM((TN, 1), jnp.float32),
        pltpu.VMEM((TN, 1), jnp.int32),
        pltpu.VMEM((TN, 1), jnp.float32),
        pltpu.VMEM((TN, D), jnp.float32),
    ],
    compiler_params=pltpu.CompilerParams(
        dimension_semantics=("arbitrary", "arbitrary")),
)


def _final_body(x_ref, r3_ref, q3_ref, hist_ref, qout_ref, loss_ref, perp_ref):
    r4 = r3_ref[...] - q3_ref[...]
    qout_ref[...] = x_ref[...] - r4
    loss_ref[0, 0] = jnp.sum(r4 * r4)
    counts = jnp.sum(hist_ref[...], axis=0)
    p = counts * (1.0 / (NQ * N))
    ent = jnp.sum(p * jnp.log(p + 1e-10))
    perp_ref[0, 0] = jnp.exp(-ent)


_final_call = pl.pallas_call(
    _final_body,
    out_shape=[
        jax.ShapeDtypeStruct((N, D), jnp.float32),
        jax.ShapeDtypeStruct((1, 1), jnp.float32),
        jax.ShapeDtypeStruct((1, 1), jnp.float32),
    ],
    out_specs=[
        pl.BlockSpec((N, D), lambda: (0, 0)),
        pl.BlockSpec(memory_space=pltpu.SMEM),
        pl.BlockSpec(memory_space=pltpu.SMEM),
    ],
)

@functools.lru_cache(maxsize=None)
def _get_sc_kernels():
    mesh = plsc.VectorSubcoreMesh(
        core_axis_name="c", subcore_axis_name="s",
        num_cores=SC_CORES, num_subcores=SC_SUBCORES)

    @functools.partial(
        pl.kernel,
        out_type=jax.ShapeDtypeStruct((N, D), jnp.float32),
        mesh=mesh,
        scratch_types=[
            pltpu.VMEM((2, GCHUNK), jnp.int32),
            pltpu.VMEM((BPW, D), jnp.float32),
            pltpu.SemaphoreType.DMA,
        ],
    )
    def _sc_gather(cb_hbm, idx_hbm, out_hbm, idx_v, rows_v, sem):
        wid = lax.axis_index("s") * SC_CORES + lax.axis_index("c")
        pltpu.sync_copy(idx_hbm.at[wid], idx_v)
        cp0 = pltpu.async_copy(cb_hbm.at[idx_v.at[0]],
                               rows_v.at[pl.ds(0, GCHUNK)], sem)
        cp1 = pltpu.async_copy(cb_hbm.at[idx_v.at[1]],
                               rows_v.at[pl.ds(GCHUNK, GCHUNK)], sem)
        cp0.wait()
        cp1.wait()
        pltpu.sync_copy(rows_v, out_hbm.at[pl.ds(wid * BPW, BPW)])

    @functools.partial(
        pl.kernel,
        out_type=jax.ShapeDtypeStruct((NW, K), jnp.float32),
        mesh=mesh,
        scratch_types=[
            pltpu.VMEM((IPW,), jnp.int32),
            pltpu.VMEM((K,), jnp.float32),
        ],
        compiler_params=pltpu.CompilerParams(needs_layout_passes=False),
    )
    def _sc_hist(idx_hbm, out_hbm, idx_v, hist_v):
        wid = lax.axis_index("s") * SC_CORES + lax.axis_index("c")
        pltpu.sync_copy(idx_hbm.at[wid], idx_v)

        zeros16 = jnp.zeros((16,), jnp.float32)

        def _zero(i, carry):
            hist_v[pl.ds(i * 16, 16)] = zeros16
            return carry

        lax.fori_loop(0, K // 16, _zero, 0)

        ones16 = jnp.ones((16,), jnp.float32)

        def _accum(i, carry):
            idx16 = idx_v[pl.ds(i * 16, 16)]
            plsc.addupdate_scatter(hist_v, [idx16], ones16)
            return carry

        lax.fori_loop(0, IPW // 16, _accum, 0)
        pltpu.sync_copy(hist_v, out_hbm.at[wid])

    return _sc_gather, _sc_hist


@jax.jit
def kernel(x, codebooks):
    sc_gather, sc_hist = _get_sc_kernels()
    flat = x.reshape(N, D)
    r = flat
    q = jnp.zeros_like(flat)
    idxs = []
    loss_sums = []
    for i in range(NQ):
        idx_i, r_i, lsum = _stage_call(r, q, codebooks[i])
        if i > 0:
            loss_sums.append(jnp.sum(lsum))
        q = sc_gather(codebooks[i], idx_i.reshape(NW, 2, GCHUNK))
        r = r_i
        idxs.append(idx_i)
    idx_all = jnp.concatenate(idxs, axis=1)
    hist = sc_hist(idx_all.reshape(NW, IPW))
    qout, lsum3, perp = _final_call(flat, r, q, hist)
    loss_sums.append(lsum3[0, 0])
    all_losses = jnp.stack(loss_sums) * (1.0 / (N * D))
    return (qout.reshape(B, T, D),
            idx_all.reshape(B, T, NQ),
            all_losses,
            perp[0, 0])
